# baseline (speedup 1.0000x reference)
import functools

import jax
import jax.numpy as jnp
from jax.experimental import pallas as pl

N = 10000
E = 320000
F = 128
R = 16

NODE_BLK = 400
EDGE_BLK = 4000


def _phi_body(sf_ref, W1_ref, b1_ref, W2_ref, b2_ref, phi_ref):
    x = sf_ref[...]
    h = jnp.dot(x, W1_ref[...], preferred_element_type=jnp.float32) + b1_ref[...]
    h = h * jax.nn.sigmoid(h)
    phi_ref[...] = (
        jnp.dot(h, W2_ref[...], preferred_element_type=jnp.float32) + b2_ref[...]
    )


def _wm_body(rbf_ref, dist_ref, Wrbf_ref, brbf_ref, cut_ref, wm_ref):
    rbf = rbf_ref[...]
    lin = jnp.dot(rbf, Wrbf_ref[...], preferred_element_type=jnp.float32) + brbf_ref[...]
    coscut = 0.5 * (1.0 + jnp.cos(jnp.pi * dist_ref[...] / cut_ref[0, 0]))
    wm_ref[...] = lin * coscut


def _dense_stage(sf, edge_rbf_pad, edge_distance2d, cut2d, W1, b1, W2, b2, Wrbf_pad, brbf):
    phi = pl.pallas_call(
        _phi_body,
        grid=(N // NODE_BLK,),
        in_specs=[
            pl.BlockSpec((NODE_BLK, F), lambda i: (i, 0)),
            pl.BlockSpec((F, F), lambda i: (0, 0)),
            pl.BlockSpec((F,), lambda i: (0,)),
            pl.BlockSpec((F, 3 * F), lambda i: (0, 0)),
            pl.BlockSpec((3 * F,), lambda i: (0,)),
        ],
        out_specs=pl.BlockSpec((NODE_BLK, 3 * F), lambda i: (i, 0)),
        out_shape=jax.ShapeDtypeStruct((N, 3 * F), jnp.float32),
    )(sf, W1, b1, W2, b2)

    wm = pl.pallas_call(
        _wm_body,
        grid=(E // EDGE_BLK,),
        in_specs=[
            pl.BlockSpec((EDGE_BLK, F), lambda i: (i, 0)),
            pl.BlockSpec((EDGE_BLK, 1), lambda i: (i, 0)),
            pl.BlockSpec((F, 3 * F), lambda i: (0, 0)),
            pl.BlockSpec((3 * F,), lambda i: (0,)),
            pl.BlockSpec((1, 1), lambda i: (0, 0)),
        ],
        out_specs=pl.BlockSpec((EDGE_BLK, 3 * F), lambda i: (i, 0)),
        out_shape=jax.ShapeDtypeStruct((E, 3 * F), jnp.float32),
    )(edge_rbf_pad, edge_distance2d, Wrbf_pad, brbf, cut2d)
    return phi, wm


def kernel(sf, vf, edge_indexes, edge_vector, edge_distance, edge_rbf, cutoff_dist,
           W1, b1, W2, b2, Wrbf, brbf):
    src = edge_indexes[0]
    nbr = edge_indexes[1]
    rbf_pad = jnp.pad(edge_rbf, ((0, 0), (0, F - R)))
    Wrbf_pad = jnp.pad(Wrbf, ((0, F - R), (0, 0)))
    cut2d = jnp.asarray(cutoff_dist, jnp.float32).reshape(1, 1)

    phi, wm = _dense_stage(sf, rbf_pad, edge_distance[:, None], cut2d,
                           W1, b1, W2, b2, Wrbf_pad, brbf)

    fm = wm * jnp.take(phi, nbr, axis=0)
    Wsf = fm[:, :F]
    Wvf_vf = fm[:, F:2 * F]
    Wvf_sf = fm[:, 2 * F:]
    dsf = jnp.zeros((N, F), jnp.float32).at[src].add(Wsf)
    ev = edge_vector / edge_distance[:, None]
    nvf = jnp.take(vf, nbr, axis=0)
    dvec = Wvf_vf[:, None, :] * nvf + ev[:, :, None] * Wvf_sf[:, None, :]
    dvf = jnp.zeros((N, 3, F), jnp.float32).at[src].add(dvec)
    return (dsf, dvf)


# speedup vs baseline: 2.7749x; 2.7749x over previous
import functools

import jax
import jax.numpy as jnp
from jax import lax
from jax.experimental import pallas as pl
from jax.experimental.pallas import tpu as pltpu
from jax.experimental.pallas import tpu_sc as plsc

N = 10000
E = 320000
F = 128
R = 16

NODE_BLK = 400
EDGE_BLK = 2560

BLK = 1280
NCHUNK = BLK // 128
NBLOCKS = E // BLK
NPAIR = BLK // 2


def _tbl_body(sf_ref, vff_ref, W1_ref, b1_ref, W2_ref, b2_ref, tbl_ref):
    x = sf_ref[...]
    h = jnp.dot(x, W1_ref[...], preferred_element_type=jnp.float32) + b1_ref[...]
    h = h * jax.nn.sigmoid(h)
    phi = jnp.dot(h, W2_ref[...], preferred_element_type=jnp.float32) + b2_ref[...]
    phi1 = phi[:, F:2 * F]
    vff = vff_ref[...]
    tbl_ref[0] = phi[:, :F]
    tbl_ref[1] = phi1 * vff[:, :F]
    tbl_ref[2] = phi1 * vff[:, F:2 * F]
    tbl_ref[3] = phi1 * vff[:, 2 * F:]
    tbl_ref[4] = phi[:, 2 * F:]


def _co_body(rbf_ref, edgeT_ref, WrbfT_ref, cut_ref, co_ref):
    x = rbf_ref[...]
    wm = lax.dot_general(WrbfT_ref[...], x, (((1,), (1,)), ((), ())),
                         preferred_element_type=jnp.float32)
    aux = edgeT_ref[...]
    d = aux[0:1, :]
    coscut = 0.5 * (1.0 + jnp.cos(jnp.pi * d / cut_ref[0, 0]))
    wm = wm * coscut
    co_ref[0:3 * F, :] = wm
    wm2_over_d = wm[2 * F:3 * F, :] * (1.0 / d)
    co_ref[3 * F:4 * F, :] = wm2_over_d * aux[1:2, :]
    co_ref[4 * F:5 * F, :] = wm2_over_d * aux[2:3, :]
    co_ref[5 * F:6 * F, :] = wm2_over_d * aux[3:4, :]


def _dense_stage(sf, vff, rbf_aug, edgeT, cut2d, W1, b1, W2, b2, WrbfT_aug):
    tbl = pl.pallas_call(
        _tbl_body,
        grid=(N // NODE_BLK,),
        in_specs=[
            pl.BlockSpec((NODE_BLK, F), lambda i: (i, 0)),
            pl.BlockSpec((NODE_BLK, 3 * F), lambda i: (i, 0)),
            pl.BlockSpec((F, F), lambda i: (0, 0)),
            pl.BlockSpec((F,), lambda i: (0,)),
            pl.BlockSpec((F, 3 * F), lambda i: (0, 0)),
            pl.BlockSpec((3 * F,), lambda i: (0,)),
        ],
        out_specs=pl.BlockSpec((5, NODE_BLK, F), lambda i: (0, i, 0)),
        out_shape=jax.ShapeDtypeStruct((5, N, F), jnp.float32),
    )(sf, vff, W1, b1, W2, b2)

    co = pl.pallas_call(
        _co_body,
        grid=(E // EDGE_BLK,),
        in_specs=[
            pl.BlockSpec((EDGE_BLK, F), lambda i: (i, 0)),
            pl.BlockSpec((8, EDGE_BLK), lambda i: (0, i)),
            pl.BlockSpec((3 * F, F), lambda i: (0, 0)),
            pl.BlockSpec((1, 1), lambda i: (0, 0)),
        ],
        out_specs=pl.BlockSpec((6 * F, EDGE_BLK), lambda i: (0, i)),
        out_shape=jax.ShapeDtypeStruct((6 * F, E), jnp.float32),
    )(rbf_aug, edgeT, WrbfT_aug, cut2d)
    return tbl, co


_MESH = plsc.VectorSubcoreMesh(core_axis_name="c", subcore_axis_name="s")


@functools.partial(
    pl.kernel,
    out_type=jax.ShapeDtypeStruct((64, N, 8), jnp.float32),
    mesh=_MESH,
    compiler_params=pltpu.CompilerParams(needs_layout_passes=False,
                                         use_tc_tiling_on_sc=False),
    scratch_types=[
        pltpu.VMEM((N, 8), jnp.float32),
        pltpu.VMEM((NCHUNK, 128), jnp.int32),
        pltpu.VMEM((BLK,), jnp.int32),
        pltpu.VMEM((BLK,), jnp.int32),
        pltpu.VMEM((BLK, 8), jnp.float32),
        pltpu.VMEM((8, BLK), jnp.float32),
        pltpu.SemaphoreType.DMA,
    ],
)
def _sc_scatter(tblf, co, nbr_hbm, src_hbm, zer, out,
                accum, idx2d, nbr1d, src_v, rows, coeff, gsem):
    wid = lax.axis_index("s") * 2 + lax.axis_index("c")
    lane = lax.iota(jnp.int32, 16)
    iota8 = lane & 7
    half = lane >> 3
    mlo = lane < 8
    mhi = lane >= 8

    def do_subterm(off, ro):
        def blk_body(b, _):
            base = b * BLK
            pltpu.sync_copy(nbr_hbm.at[pl.ds(base, BLK)], nbr1d)
            pltpu.sync_copy(src_hbm.at[pl.ds(base, BLK)], src_v)
            pltpu.sync_copy(co.at[pl.ds(ro, 8), pl.ds(base, BLK)], coeff)

            def adj(i, _):
                t = i // 8
                k = i % 8
                v = nbr1d[pl.ds(128 * t + 16 * k, 16)]
                idx2d[t, pl.ds(16 * k, 16)] = v * 16 + off
                return 0
            lax.fori_loop(0, NCHUNK * 8, adj, 0, unroll=4)

            cps = [
                pltpu.async_copy(tblf.at[idx2d.at[q]],
                                 rows.at[pl.ds(q * 128, 128)], gsem)
                for q in range(NCHUNK)
            ]
            for cp in cps:
                cp.wait()

            def pair(p, idx16):
                sp = plsc.load_gather(src_v, [idx16])
                rv = plsc.load_gather(rows, [idx16, iota8])
                cf = plsc.load_gather(coeff, [iota8, idx16])
                prod = rv * cf
                plsc.addupdate_scatter(accum, [sp, iota8], prod, mask=mlo)
                plsc.addupdate_scatter(accum, [sp, iota8], prod, mask=mhi)
                return idx16 + 2
            lax.fori_loop(0, NPAIR, pair, half, unroll=4)
            return 0
        lax.fori_loop(0, NBLOCKS, blk_body, 0)

    for r in range(2):
        a = r * 32 + wid
        is_dsf = a < 16
        am16 = jnp.maximum(a - 16, 0)
        c = am16 // 16
        j = am16 % 16
        plane_a = jnp.where(is_dsf, 0, 1 + c)
        j_a = jnp.where(is_dsf, a, j)
        off_a = plane_a * (N * 16) + j_a
        ro_a = jnp.where(is_dsf, 8 * a, F + 8 * j)

        pltpu.sync_copy(zer, accum)
        do_subterm(off_a, ro_a)

        @pl.when(jnp.logical_not(is_dsf))
        def _():
            off_b = 4 * (N * 16) + j
            ro_b = 3 * F + c * F + 8 * j
            do_subterm(off_b, ro_b)

        pltpu.sync_copy(accum, out.at[a])


def kernel(sf, vf, edge_indexes, edge_vector, edge_distance, edge_rbf, cutoff_dist,
           W1, b1, W2, b2, Wrbf, brbf):
    src = edge_indexes[0]
    nbr = edge_indexes[1]

    rbf_aug = jnp.concatenate(
        [edge_rbf, jnp.ones((E, 1), jnp.float32), jnp.zeros((E, F - R - 1), jnp.float32)],
        axis=1)
    edgeT = jnp.concatenate(
        [edge_distance.reshape(1, E), edge_vector.T, jnp.zeros((4, E), jnp.float32)],
        axis=0)
    WrbfT_aug = jnp.concatenate(
        [Wrbf, brbf[None, :], jnp.zeros((F - R - 1, 3 * F), jnp.float32)], axis=0).T
    cut2d = jnp.asarray(cutoff_dist, jnp.float32).reshape(1, 1)
    vff = vf.reshape(N, 3 * F)

    tbl, co = _dense_stage(sf, vff, rbf_aug, edgeT, cut2d, W1, b1, W2, b2, WrbfT_aug)

    tblf = tbl.reshape(5 * N * 16, 8)
    zer = jnp.zeros((N, 8), jnp.float32)

    out = _sc_scatter(tblf, co, nbr, src, zer)

    dsf = out[:16].transpose(1, 0, 2).reshape(N, F)
    dvf = out[16:].reshape(3, 16, N, 8).transpose(2, 0, 1, 3).reshape(N, 3, F)
    return (dsf, dvf)


# speedup vs baseline: 8.9823x; 3.2370x over previous
import functools

import jax
import jax.numpy as jnp
from jax import lax
from jax.experimental import pallas as pl
from jax.experimental.pallas import tpu as pltpu
from jax.experimental.pallas import tpu_sc as plsc

N = 10000
E = 320000
F = 128
R = 16

NODE_BLK = 400
EDGE_BLK = 2560

BLK = 640
NCHUNK = BLK // 128
NBLOCKS = E // BLK
NPAIR = BLK // 2
NSET = 4


def _tbl_body(sf_ref, vff_ref, W1_ref, b1_ref, W2_ref, b2_ref, tbl_ref):
    x = sf_ref[...]
    h = jnp.dot(x, W1_ref[...], preferred_element_type=jnp.float32) + b1_ref[...]
    h = h * jax.nn.sigmoid(h)
    phi = jnp.dot(h, W2_ref[...], preferred_element_type=jnp.float32) + b2_ref[...]
    phi1 = phi[:, F:2 * F]
    vff = vff_ref[...]
    tbl_ref[0] = phi[:, :F]
    tbl_ref[1] = phi1 * vff[:, :F]
    tbl_ref[2] = phi1 * vff[:, F:2 * F]
    tbl_ref[3] = phi1 * vff[:, 2 * F:]
    tbl_ref[4] = phi[:, 2 * F:]


def _co_body(rbf_ref, edgeT_ref, WrbfT_ref, brbfT_ref, cut_ref, co_ref):
    x = rbf_ref[...]
    wm = lax.dot_general(WrbfT_ref[...], x, (((1,), (1,)), ((), ())),
                         preferred_element_type=jnp.float32) + brbfT_ref[...]
    aux = edgeT_ref[...]
    d = aux[0:1, :]
    coscut = 0.5 * (1.0 + jnp.cos(jnp.pi * d / cut_ref[0, 0]))
    wm = wm * coscut
    co_ref[0:2 * F, :] = wm[0:2 * F, :]
    wm2_over_d = wm[2 * F:3 * F, :] * (1.0 / d)
    co_ref[2 * F:3 * F, :] = wm2_over_d * aux[1:2, :]
    co_ref[3 * F:4 * F, :] = wm2_over_d * aux[2:3, :]
    co_ref[4 * F:5 * F, :] = wm2_over_d * aux[3:4, :]


def _dense_stage(sf, vff, rbf, edgeT, cut2d, W1, b1, W2, b2, WrbfT, brbfT):
    tbl = pl.pallas_call(
        _tbl_body,
        grid=(N // NODE_BLK,),
        in_specs=[
            pl.BlockSpec((NODE_BLK, F), lambda i: (i, 0)),
            pl.BlockSpec((NODE_BLK, 3 * F), lambda i: (i, 0)),
            pl.BlockSpec((F, F), lambda i: (0, 0)),
            pl.BlockSpec((F,), lambda i: (0,)),
            pl.BlockSpec((F, 3 * F), lambda i: (0, 0)),
            pl.BlockSpec((3 * F,), lambda i: (0,)),
        ],
        out_specs=pl.BlockSpec((5, NODE_BLK, F), lambda i: (0, i, 0)),
        out_shape=jax.ShapeDtypeStruct((5, N, F), jnp.float32),
    )(sf, vff, W1, b1, W2, b2)

    co = pl.pallas_call(
        _co_body,
        grid=(E // EDGE_BLK,),
        in_specs=[
            pl.BlockSpec((EDGE_BLK, R), lambda i: (i, 0)),
            pl.BlockSpec((8, EDGE_BLK), lambda i: (0, i)),
            pl.BlockSpec((3 * F, R), lambda i: (0, 0)),
            pl.BlockSpec((3 * F, 1), lambda i: (0, 0)),
            pl.BlockSpec((1, 1), lambda i: (0, 0)),
        ],
        out_specs=pl.BlockSpec((5 * F, EDGE_BLK), lambda i: (0, i)),
        out_shape=jax.ShapeDtypeStruct((5 * F, E), jnp.float32),
    )(rbf, edgeT, WrbfT, brbfT, cut2d)
    return tbl, co


_MESH = plsc.VectorSubcoreMesh(core_axis_name="c", subcore_axis_name="s")


@functools.partial(
    pl.kernel,
    out_type=(jax.ShapeDtypeStruct((N, F), jnp.float32),
              jax.ShapeDtypeStruct((N, 3 * F), jnp.float32)),
    mesh=_MESH,
    compiler_params=pltpu.CompilerParams(needs_layout_passes=False,
                                         use_tc_tiling_on_sc=False),
    scratch_types=[
        pltpu.VMEM((N, 8), jnp.float32),
        pltpu.VMEM((NSET, BLK), jnp.int32),
        pltpu.VMEM((NSET, BLK), jnp.int32),
        pltpu.VMEM((NSET, BLK, 8), jnp.float32),
        pltpu.VMEM((NSET, 8, BLK), jnp.float32),
        pltpu.SemaphoreType.DMA((NSET,)),
        pltpu.SemaphoreType.DMA((NSET,)),
    ],
)
def _sc_scatter(tblf, co, nbr_hbm, src_hbm, zer, dsf_out, dvf_out,
                accum, nbr3, src3, rows3, coeff3, lsem, gsem):
    wid = lax.axis_index("s") * 2 + lax.axis_index("c")
    lane = lax.iota(jnp.int32, 16)
    iota8 = lane & 7
    half = lane >> 3
    mlo = lane < 8
    mhi = lane >= 8

    def do_subterm(off, ro):
        def linear_copies(b):
            m = b % NSET
            base = b * BLK
            return [
                pltpu.make_async_copy(nbr_hbm.at[pl.ds(base, BLK)],
                                      nbr3.at[m], lsem.at[m]),
                pltpu.make_async_copy(src_hbm.at[pl.ds(base, BLK)],
                                      src3.at[m], lsem.at[m]),
                pltpu.make_async_copy(co.at[pl.ds(ro, 8), pl.ds(base, BLK)],
                                      coeff3.at[m], lsem.at[m]),
            ]

        def gather_copies(b):
            m = b % NSET
            return [
                pltpu.make_async_copy(
                    tblf.at[nbr3.at[m, pl.ds(q * 128, 128)]],
                    rows3.at[m, pl.ds(q * 128, 128)], gsem.at[m])
                for q in range(NCHUNK)
            ]

        def prefetch(b):
            m = b % NSET
            for cp in linear_copies(b):
                cp.wait()

            @plsc.parallel_loop(0, BLK // 16, 1, unroll=8)
            def _adj(i):
                v = nbr3[m, pl.ds(16 * i, 16)]
                nbr3[m, pl.ds(16 * i, 16)] = v * 16 + off
            for cp in gather_copies(b):
                cp.start()

        for cp in linear_copies(0):
            cp.start()
        prefetch(0)
        for cp in linear_copies(1):
            cp.start()
        prefetch(1)
        for cp in linear_copies(2):
            cp.start()

        def blk_body(b, _):
            @pl.when(b + 3 < NBLOCKS)
            def _():
                for cp in linear_copies(b + 3):
                    cp.start()

            @pl.when(b + 2 < NBLOCKS)
            def _():
                prefetch(b + 2)

            for cp in gather_copies(b):
                cp.wait()

            m = b % NSET
            rowsm = rows3.at[m]
            coeffm = coeff3.at[m]
            srcm = src3.at[m]

            @plsc.parallel_loop(0, NPAIR, 1, unroll=8, carry=half)
            def _pair(p, idx16):
                sp = plsc.load_gather(srcm, [idx16])
                rv = plsc.load_gather(rowsm, [idx16, iota8])
                cf = plsc.load_gather(coeffm, [iota8, idx16])
                prod = rv * cf
                plsc.addupdate_scatter(accum, [sp, iota8], prod, mask=mlo)
                plsc.addupdate_scatter(accum, [sp, iota8], prod, mask=mhi)
                return idx16 + 2
            return 0
        lax.fori_loop(0, NBLOCKS, blk_body, 0)

    for r in range(2):
        a = r * 32 + wid
        is_dsf = a < 16
        am16 = jnp.maximum(a - 16, 0)
        c = am16 // 16
        j = am16 % 16
        plane_a = jnp.where(is_dsf, 0, 1 + c)
        j_a = jnp.where(is_dsf, a, j)
        off_a = plane_a * (N * 16) + j_a
        ro_a = jnp.where(is_dsf, 8 * a, F + 8 * j)

        pltpu.sync_copy(zer, accum)
        do_subterm(off_a, ro_a)

        @pl.when(jnp.logical_not(is_dsf))
        def _():
            off_b = 4 * (N * 16) + j
            ro_b = 2 * F + c * F + 8 * j
            do_subterm(off_b, ro_b)

        @pl.when(is_dsf)
        def _():
            pltpu.sync_copy(accum, dsf_out.at[:, pl.ds(8 * a, 8)])

        @pl.when(jnp.logical_not(is_dsf))
        def _():
            pltpu.sync_copy(accum, dvf_out.at[:, pl.ds(c * F + 8 * j, 8)])


def kernel(sf, vf, edge_indexes, edge_vector, edge_distance, edge_rbf, cutoff_dist,
           W1, b1, W2, b2, Wrbf, brbf):
    src = edge_indexes[0]
    nbr = edge_indexes[1]

    edgeT = jnp.concatenate(
        [edge_distance.reshape(1, E), edge_vector.T, jnp.zeros((4, E), jnp.float32)],
        axis=0)
    WrbfT = Wrbf.T
    brbfT = brbf[:, None]
    cut2d = jnp.asarray(cutoff_dist, jnp.float32).reshape(1, 1)
    vff = vf.reshape(N, 3 * F)

    tbl, co = _dense_stage(sf, vff, edge_rbf, edgeT, cut2d, W1, b1, W2, b2, WrbfT, brbfT)

    tblf = tbl.reshape(5 * N * 16, 8)
    zer = jnp.zeros((N, 8), jnp.float32)

    dsf, dvf2 = _sc_scatter(tblf, co, nbr, src, zer)
    return (dsf, dvf2.reshape(N, 3, F))
